# baseline (speedup 1.0000x reference)
import functools

import jax
import jax.numpy as jnp
from jax import lax
from jax.experimental import pallas as pl
from jax.experimental.pallas import tpu as pltpu
from jax.experimental.pallas import tpu_sc as plsc

N_ROWS = 100000
D = 64
B = 4096 * 200
H = B // 2
NC, NS = 2, 16
NW = NC * NS
PER_W = H // NW
CHUNK = 160
NBUF = 4
N_CHUNKS = PER_W // CHUNK
assert PER_W % CHUNK == 0 and N_CHUNKS % NBUF == 0


def _make_kernel():
    mesh = plsc.VectorSubcoreMesh(core_axis_name="c", subcore_axis_name="s")

    @functools.partial(
        pl.kernel,
        out_type=jax.ShapeDtypeStruct((H, 2 * D), jnp.float32),
        mesh=mesh,
        scratch_types=(
            [pltpu.VMEM((NBUF, CHUNK), jnp.int32),
             pltpu.VMEM((NBUF, CHUNK), jnp.int32),
             pltpu.VMEM((NBUF, CHUNK, D), jnp.float32),
             pltpu.VMEM((NBUF, CHUNK, D), jnp.float32)]
            + [pltpu.SemaphoreType.DMA] * (3 * NBUF)
        ),
        compiler_params=pltpu.CompilerParams(use_tc_tiling_on_sc=False),
    )
    def gather_kernel(idxe_hbm, idxo_hbm, table_hbm, out_hbm,
                      idxe_v, idxo_v, rowse_v, rowso_v, *sems):
        isem = sems[0:NBUF]
        gsem = sems[NBUF:2 * NBUF]
        osem = sems[2 * NBUF:3 * NBUF]
        wid = lax.axis_index("s") * NC + lax.axis_index("c")
        w_base = wid * PER_W

        def start_idx_load(c, b):
            pltpu.async_copy(idxe_hbm.at[pl.ds(w_base + c * CHUNK, CHUNK)],
                             idxe_v.at[b], isem[b])
            pltpu.async_copy(idxo_hbm.at[pl.ds(w_base + c * CHUNK, CHUNK)],
                             idxo_v.at[b], isem[b])

        def wait_idx_load(b):
            pltpu.make_async_copy(idxe_hbm.at[pl.ds(w_base, CHUNK)],
                                  idxe_v.at[b], isem[b]).wait()
            pltpu.make_async_copy(idxo_hbm.at[pl.ds(w_base, CHUNK)],
                                  idxo_v.at[b], isem[b]).wait()

        def start_gather(b):
            pltpu.async_copy(table_hbm.at[idxe_v.at[b]], rowse_v.at[b],
                             gsem[b])
            pltpu.async_copy(table_hbm.at[idxo_v.at[b]], rowso_v.at[b],
                             gsem[b])

        def wait_gather(b):
            pltpu.make_async_copy(table_hbm.at[idxe_v.at[b]], rowse_v.at[b],
                                  gsem[b]).wait()
            pltpu.make_async_copy(table_hbm.at[idxo_v.at[b]], rowso_v.at[b],
                                  gsem[b]).wait()

        def start_store(c, b):
            base = w_base + c * CHUNK
            pltpu.async_copy(rowse_v.at[b],
                             out_hbm.at[pl.ds(base, CHUNK), pl.ds(0, D)],
                             osem[b])
            pltpu.async_copy(rowso_v.at[b],
                             out_hbm.at[pl.ds(base, CHUNK), pl.ds(D, D)],
                             osem[b])

        def wait_store(b):
            pltpu.make_async_copy(rowse_v.at[b],
                                  out_hbm.at[pl.ds(w_base, CHUNK),
                                             pl.ds(0, D)],
                                  osem[b]).wait()
            pltpu.make_async_copy(rowso_v.at[b],
                                  out_hbm.at[pl.ds(w_base, CHUNK),
                                             pl.ds(D, D)],
                                  osem[b]).wait()

        for b in range(NBUF):
            start_idx_load(b, b)

        def body(g, carry):
            for b in range(NBUF):
                i = g * NBUF + b
                j = i - (NBUF - 1)
                sj = (b + 1) % NBUF

                @pl.when(j >= 0)
                def _():
                    wait_gather(sj)
                    start_store(j, sj)

                    @pl.when(j + NBUF < N_CHUNKS)
                    def _():
                        start_idx_load(j + NBUF, sj)

                @pl.when(i >= NBUF)
                def _():
                    wait_store(b)

                wait_idx_load(b)
                start_gather(b)
            return carry

        lax.fori_loop(0, N_CHUNKS // NBUF, body, 0)

        for j in range(N_CHUNKS - NBUF + 1, N_CHUNKS):
            wait_gather(j % NBUF)
            start_store(j, j % NBUF)
        for j in range(N_CHUNKS - NBUF, N_CHUNKS):
            wait_store(j % NBUF)

    return gather_kernel


_gather = _make_kernel()


def kernel(t, pe):
    idx2 = t.reshape(-1, 2).astype(jnp.int32)
    return _gather(idx2[:, 0], idx2[:, 1], pe).reshape(B, D)


# speedup vs baseline: 1.0622x; 1.0622x over previous
import functools

import jax
import jax.numpy as jnp
from jax import lax
from jax.experimental import pallas as pl
from jax.experimental.pallas import tpu as pltpu
from jax.experimental.pallas import tpu_sc as plsc

N_ROWS = 100000
D = 64
B = 4096 * 200
NC, NS = 2, 16
NW = NC * NS
PER_W = B // NW
CHUNK = 400
NBUF = 4
N_CHUNKS = PER_W // CHUNK
assert PER_W % CHUNK == 0 and N_CHUNKS % NBUF == 0

BM = 2048


def _make_gather():
    mesh = plsc.VectorSubcoreMesh(core_axis_name="c", subcore_axis_name="s")

    @functools.partial(
        pl.kernel,
        out_type=jax.ShapeDtypeStruct((B, 2 * D), jnp.float32),
        mesh=mesh,
        scratch_types=(
            [pltpu.VMEM((NBUF, CHUNK), jnp.int32),
             pltpu.VMEM((NBUF, CHUNK, D), jnp.float32)]
            + [pltpu.SemaphoreType.DMA] * (3 * NBUF)
        ),
        compiler_params=pltpu.CompilerParams(use_tc_tiling_on_sc=False),
    )
    def gather_kernel(idx_hbm, table_hbm, out_hbm, idx_v, rows_v, *sems):
        isem = sems[0:NBUF]
        gsem = sems[NBUF:2 * NBUF]
        osem = sems[2 * NBUF:3 * NBUF]
        wid = lax.axis_index("s") * NC + lax.axis_index("c")
        w_base = wid * PER_W

        def start_idx_load(c, b):
            pltpu.async_copy(idx_hbm.at[pl.ds(w_base + c * CHUNK, CHUNK)],
                             idx_v.at[b], isem[b])

        def wait_idx_load(b):
            pltpu.make_async_copy(idx_hbm.at[pl.ds(w_base, CHUNK)],
                                  idx_v.at[b], isem[b]).wait()

        def start_gather(b):
            pltpu.async_copy(table_hbm.at[idx_v.at[b]], rows_v.at[b], gsem[b])

        def wait_gather(b):
            pltpu.make_async_copy(table_hbm.at[idx_v.at[b]], rows_v.at[b],
                                  gsem[b]).wait()

        def start_store(c, b):
            pltpu.async_copy(rows_v.at[b],
                             out_hbm.at[pl.ds(w_base + c * CHUNK, CHUNK),
                                        pl.ds(0, D)],
                             osem[b])

        def wait_store(b):
            pltpu.make_async_copy(rows_v.at[b],
                                  out_hbm.at[pl.ds(w_base, CHUNK),
                                             pl.ds(0, D)],
                                  osem[b]).wait()

        for b in range(NBUF):
            start_idx_load(b, b)

        def body(g, carry):
            for b in range(NBUF):
                i = g * NBUF + b
                j = i - (NBUF - 1)
                sj = (b + 1) % NBUF

                @pl.when(j >= 0)
                def _():
                    wait_gather(sj)
                    start_store(j, sj)

                    @pl.when(j + NBUF < N_CHUNKS)
                    def _():
                        start_idx_load(j + NBUF, sj)

                @pl.when(i >= NBUF)
                def _():
                    wait_store(b)

                wait_idx_load(b)
                start_gather(b)
            return carry

        lax.fori_loop(0, N_CHUNKS // NBUF, body, 0)

        for j in range(N_CHUNKS - NBUF + 1, N_CHUNKS):
            wait_gather(j % NBUF)
            start_store(j, j % NBUF)
        for j in range(N_CHUNKS - NBUF, N_CHUNKS):
            wait_store(j % NBUF)

    return gather_kernel


def _copy_body(y_ref, o_ref):
    o_ref[...] = y_ref[:, :D]


_take_left = pl.pallas_call(
    _copy_body,
    grid=(B // BM,),
    in_specs=[pl.BlockSpec((BM, 2 * D), lambda m: (m, 0))],
    out_specs=pl.BlockSpec((BM, D), lambda m: (m, 0)),
    out_shape=jax.ShapeDtypeStruct((B, D), jnp.float32),
)

_gather = _make_gather()


def kernel(t, pe):
    idx = t.reshape(-1).astype(jnp.int32)
    wide = _gather(idx, pe)
    return _take_left(wide)


# speedup vs baseline: 2.4079x; 2.2669x over previous
import functools

import jax
import jax.numpy as jnp
from jax import lax
from jax.experimental import pallas as pl
from jax.experimental.pallas import tpu as pltpu
from jax.experimental.pallas import tpu_sc as plsc

N_ROWS = 100000
D = 64
B = 4096 * 200
NC, NS = 2, 16
NW = NC * NS
PER_W = B // NW
CHUNK = 400
NBUF = 4
N_CHUNKS = PER_W // CHUNK
assert PER_W % CHUNK == 0 and N_CHUNKS % NBUF == 0

BM = 2048


def _make_gather():
    mesh = plsc.VectorSubcoreMesh(core_axis_name="c", subcore_axis_name="s")

    @functools.partial(
        pl.kernel,
        out_type=jax.ShapeDtypeStruct((B, 2 * D), jnp.float32),
        mesh=mesh,
        scratch_types=(
            [pltpu.VMEM((NBUF, CHUNK), jnp.int32),
             pltpu.VMEM((NBUF, CHUNK, D), jnp.float32)]
            + [pltpu.SemaphoreType.DMA] * (3 * NBUF)
        ),
        compiler_params=pltpu.CompilerParams(use_tc_tiling_on_sc=False),
    )
    def gather_kernel(idx_hbm, table_hbm, out_hbm, idx_v, rows_v, *sems):
        isem = sems[0:NBUF]
        gsem = sems[NBUF:2 * NBUF]
        osem = sems[2 * NBUF:3 * NBUF]
        wid = lax.axis_index("s") * NC + lax.axis_index("c")
        w_base = wid * PER_W

        def start_idx_load(c, b):
            pltpu.async_copy(idx_hbm.at[pl.ds(w_base + c * CHUNK, CHUNK)],
                             idx_v.at[b], isem[b])

        def wait_idx_load(b):
            pltpu.make_async_copy(idx_hbm.at[pl.ds(w_base, CHUNK)],
                                  idx_v.at[b], isem[b]).wait()

        def start_gather(b):
            pltpu.async_copy(table_hbm.at[idx_v.at[b]], rows_v.at[b], gsem[b])

        def wait_gather(b):
            pltpu.make_async_copy(table_hbm.at[idx_v.at[b]], rows_v.at[b],
                                  gsem[b]).wait()

        def start_store(c, b):
            pltpu.async_copy(rows_v.at[b],
                             out_hbm.at[pl.ds(w_base + c * CHUNK, CHUNK),
                                        pl.ds(0, D)],
                             osem[b])

        def wait_store(b):
            pltpu.make_async_copy(rows_v.at[b],
                                  out_hbm.at[pl.ds(w_base, CHUNK),
                                             pl.ds(0, D)],
                                  osem[b]).wait()

        for b in range(NBUF):
            start_idx_load(b, b)

        def body(g, carry):
            for b in range(NBUF):
                i = g * NBUF + b
                j = i - (NBUF - 1)
                sj = (b + 1) % NBUF

                @pl.when(j >= 0)
                def _():
                    wait_gather(sj)
                    start_store(j, sj)

                    @pl.when(j + NBUF < N_CHUNKS)
                    def _():
                        start_idx_load(j + NBUF, sj)

                @pl.when(i >= NBUF)
                def _():
                    wait_store(b)

                wait_idx_load(b)
                start_gather(b)
            return carry

        lax.fori_loop(0, N_CHUNKS // NBUF, body, 0)

        for j in range(N_CHUNKS - NBUF + 1, N_CHUNKS):
            wait_gather(j % NBUF)
            start_store(j, j % NBUF)
        for j in range(N_CHUNKS - NBUF, N_CHUNKS):
            wait_store(j % NBUF)

    return gather_kernel


def _copy_body(y_ref, o_ref):
    o_ref[...] = y_ref[:, :D]


_take_left = pl.pallas_call(
    _copy_body,
    grid=(B // BM,),
    in_specs=[pl.BlockSpec((BM, 2 * D), lambda m: (m, 0))],
    out_specs=pl.BlockSpec((BM, D), lambda m: (m, 0)),
    out_shape=jax.ShapeDtypeStruct((B, D), jnp.float32),
)

_gather = _make_gather()


def kernel(t, pe):
    idx = t.reshape(-1).astype(jnp.int32)
    wide = _gather(idx, pe)
    return wide[:, :D]


# speedup vs baseline: 2.4143x; 1.0026x over previous
import functools

import jax
import jax.numpy as jnp
from jax import lax
from jax.experimental import pallas as pl
from jax.experimental.pallas import tpu as pltpu
from jax.experimental.pallas import tpu_sc as plsc

N_ROWS = 100000
D = 64
NR, NT = 4096, 200
B = NR * NT
NC, NS = 2, 16
NW = NC * NS
PER_W = B // NW
CHUNK = 256
NBUF = 4
N_CHUNKS = PER_W // CHUNK
assert PER_W % CHUNK == 0 and N_CHUNKS % NBUF == 0
assert NR % CHUNK == 0

BM = 2048


def _make_gather():
    mesh = plsc.VectorSubcoreMesh(core_axis_name="c", subcore_axis_name="s")

    @functools.partial(
        pl.kernel,
        out_type=jax.ShapeDtypeStruct((NR, NT, 2 * D), jnp.float32),
        mesh=mesh,
        scratch_types=(
            [pltpu.VMEM((NBUF, CHUNK), jnp.int32),
             pltpu.VMEM((NBUF, CHUNK, D), jnp.float32)]
            + [pltpu.SemaphoreType.DMA] * (3 * NBUF)
        ),
        compiler_params=pltpu.CompilerParams(use_tc_tiling_on_sc=False),
    )
    def gather_kernel(idx_hbm, table_hbm, out_hbm, idx_v, rows_v, *sems):
        isem = sems[0:NBUF]
        gsem = sems[NBUF:2 * NBUF]
        osem = sems[2 * NBUF:3 * NBUF]
        wid = lax.axis_index("s") * NC + lax.axis_index("c")
        w_base = wid * PER_W

        def start_idx_load(c, b):
            pltpu.async_copy(idx_hbm.at[pl.ds(w_base + c * CHUNK, CHUNK)],
                             idx_v.at[b], isem[b])

        def wait_idx_load(b):
            pltpu.make_async_copy(idx_hbm.at[pl.ds(w_base, CHUNK)],
                                  idx_v.at[b], isem[b]).wait()

        def start_gather(b):
            pltpu.async_copy(table_hbm.at[idx_v.at[b]], rows_v.at[b], gsem[b])

        def wait_gather(b):
            pltpu.make_async_copy(table_hbm.at[idx_v.at[b]], rows_v.at[b],
                                  gsem[b]).wait()

        def start_store(c, b):
            q0 = w_base + c * CHUNK
            col = q0 // NR
            r0 = q0 - col * NR
            pltpu.async_copy(rows_v.at[b],
                             out_hbm.at[pl.ds(r0, CHUNK), col, pl.ds(0, D)],
                             osem[b])

        def wait_store(b):
            pltpu.make_async_copy(rows_v.at[b],
                                  out_hbm.at[pl.ds(0, CHUNK), 0, pl.ds(0, D)],
                                  osem[b]).wait()

        for b in range(NBUF):
            start_idx_load(b, b)

        def body(g, carry):
            for b in range(NBUF):
                i = g * NBUF + b
                j = i - (NBUF - 1)
                sj = (b + 1) % NBUF

                @pl.when(j >= 0)
                def _():
                    wait_gather(sj)
                    start_store(j, sj)

                    @pl.when(j + NBUF < N_CHUNKS)
                    def _():
                        start_idx_load(j + NBUF, sj)

                @pl.when(i >= NBUF)
                def _():
                    wait_store(b)

                wait_idx_load(b)
                start_gather(b)
            return carry

        lax.fori_loop(0, N_CHUNKS // NBUF, body, 0)

        for j in range(N_CHUNKS - NBUF + 1, N_CHUNKS):
            wait_gather(j % NBUF)
            start_store(j, j % NBUF)
        for j in range(N_CHUNKS - NBUF, N_CHUNKS):
            wait_store(j % NBUF)

    return gather_kernel


def _copy_body(y_ref, o_ref):
    o_ref[...] = y_ref[:, :D]


_take_left = pl.pallas_call(
    _copy_body,
    grid=(B // BM,),
    in_specs=[pl.BlockSpec((BM, 2 * D), lambda m: (m, 0))],
    out_specs=pl.BlockSpec((BM, D), lambda m: (m, 0)),
    out_shape=jax.ShapeDtypeStruct((B, D), jnp.float32),
)

_gather = _make_gather()


def kernel(t, pe):
    idx_cm = t.T.reshape(-1).astype(jnp.int32)
    wide = _gather(idx_cm, pe)
    return wide.reshape(B, 2 * D)[:, :D]
